# baseline (speedup 1.0000x reference)
import jax
import jax.numpy as jnp
from jax.experimental import pallas as pl
from jax.experimental.pallas import tpu as pltpu

_DIM = 128
_K = 65536
_BATCH = 4096
_BLK = 2048


def _body(ptr_ref, keys_ref, queue_ref, out_ref):
    i = pl.program_id(0)
    base = i * _BLK
    ptr = ptr_ref[0]
    covered = (base >= ptr) & (base < ptr + _BATCH)

    @pl.when(covered)
    def _():
        off = jnp.clip(base - ptr, 0, _BATCH - _BLK)
        out_ref[...] = keys_ref[pl.ds(off, _BLK), :].T

    @pl.when(jnp.logical_not(covered))
    def _():
        out_ref[...] = queue_ref[...]


def kernel(keys, queue, queue_ptr):
    ptr = jnp.asarray(queue_ptr, jnp.int32).reshape((1,))
    new_queue = pl.pallas_call(
        _body,
        grid=(_K // _BLK,),
        in_specs=[
            pl.BlockSpec(memory_space=pltpu.SMEM),
            pl.BlockSpec((_BATCH, _DIM), lambda i: (0, 0)),
            pl.BlockSpec((_DIM, _BLK), lambda i: (0, i)),
        ],
        out_specs=pl.BlockSpec((_DIM, _BLK), lambda i: (0, i)),
        out_shape=jax.ShapeDtypeStruct((_DIM, _K), jnp.float32),
    )(ptr, keys, queue)
    new_ptr = (jnp.asarray(queue_ptr, jnp.int32) + _BATCH) % _K
    return new_queue, jnp.asarray(new_ptr, dtype=jnp.int64)


# speedup vs baseline: 1.4601x; 1.4601x over previous
import jax
import jax.numpy as jnp
from jax.experimental import pallas as pl
from jax.experimental.pallas import tpu as pltpu

_DIM = 128
_K = 65536
_BATCH = 4096
_BLK = 8192


def _body(ptr_ref, keys_ref, queue_ref, out_ref):
    i = pl.program_id(0)
    base = i * _BLK
    ptr = ptr_ref[0]
    for s in range(_BLK // _BATCH):
        lo = s * _BATCH
        covered = (base + lo) == ptr

        @pl.when(covered)
        def _():
            out_ref[:, pl.ds(lo, _BATCH)] = keys_ref[...].T

        @pl.when(jnp.logical_not(covered))
        def _():
            out_ref[:, pl.ds(lo, _BATCH)] = queue_ref[:, pl.ds(lo, _BATCH)]


def kernel(keys, queue, queue_ptr):
    ptr = jnp.asarray(queue_ptr, jnp.int32).reshape((1,))
    new_queue = pl.pallas_call(
        _body,
        grid=(_K // _BLK,),
        in_specs=[
            pl.BlockSpec(memory_space=pltpu.SMEM),
            pl.BlockSpec((_BATCH, _DIM), lambda i: (0, 0)),
            pl.BlockSpec((_DIM, _BLK), lambda i: (0, i)),
        ],
        out_specs=pl.BlockSpec((_DIM, _BLK), lambda i: (0, i)),
        out_shape=jax.ShapeDtypeStruct((_DIM, _K), jnp.float32),
    )(ptr, keys, queue)
    new_ptr = (jnp.asarray(queue_ptr, jnp.int32) + _BATCH) % _K
    return new_queue, jnp.asarray(new_ptr, dtype=jnp.int64)


# speedup vs baseline: 1.5775x; 1.0804x over previous
import jax
import jax.numpy as jnp
from jax.experimental import pallas as pl
from jax.experimental.pallas import tpu as pltpu

_DIM = 128
_K = 65536
_BATCH = 4096
_BLK = 16384


def _body(ptr_ref, keys_ref, queue_ref, out_ref):
    i = pl.program_id(0)
    base = i * _BLK
    ptr = ptr_ref[0]
    for s in range(_BLK // _BATCH):
        lo = s * _BATCH
        covered = (base + lo) == ptr

        @pl.when(covered)
        def _():
            out_ref[:, pl.ds(lo, _BATCH)] = keys_ref[...].T

        @pl.when(jnp.logical_not(covered))
        def _():
            out_ref[:, pl.ds(lo, _BATCH)] = queue_ref[:, pl.ds(lo, _BATCH)]


def kernel(keys, queue, queue_ptr):
    ptr = jnp.asarray(queue_ptr, jnp.int32).reshape((1,))
    new_queue = pl.pallas_call(
        _body,
        grid=(_K // _BLK,),
        in_specs=[
            pl.BlockSpec(memory_space=pltpu.SMEM),
            pl.BlockSpec((_BATCH, _DIM), lambda i: (0, 0)),
            pl.BlockSpec((_DIM, _BLK), lambda i: (0, i)),
        ],
        out_specs=pl.BlockSpec((_DIM, _BLK), lambda i: (0, i)),
        out_shape=jax.ShapeDtypeStruct((_DIM, _K), jnp.float32),
    )(ptr, keys, queue)
    new_ptr = (jnp.asarray(queue_ptr, jnp.int32) + _BATCH) % _K
    return new_queue, jnp.asarray(new_ptr, dtype=jnp.int64)
